# baseline (speedup 1.0000x reference)
import functools

import jax
import jax.numpy as jnp
from jax import lax
from jax.experimental import pallas as pl
from jax.experimental.pallas import tpu as pltpu
from jax.experimental.pallas import tpu_sc as plsc

_NROWS = 8
_HIDDEN = 1024
_B = 4 * 4096
_NC, _NS = 2, 16
_NW = _NC * _NS
_BPW = _B // _NW
_CH = 32
_NCHUNK = _BPW // _CH

_mesh = plsc.VectorSubcoreMesh(core_axis_name="c", subcore_axis_name="s")


@functools.partial(
    pl.kernel,
    mesh=_mesh,
    out_type=jax.ShapeDtypeStruct((_B, _HIDDEN), jnp.float32),
    scratch_types=[
        pltpu.VMEM((_BPW,), jnp.int32),
        pltpu.VMEM_SHARED((_NROWS, _HIDDEN), jnp.float32),
        pltpu.SemaphoreType.DMA,
    ],
)
def _sc_gather(idx_hbm, table_hbm, out_hbm, idx_v, table_sh, sem):
    wid = lax.axis_index("s") * _NC + lax.axis_index("c")
    base = wid * _BPW

    @pl.when(lax.axis_index("s") == 0)
    def _():
        pltpu.sync_copy(table_hbm, table_sh)

    pltpu.sync_copy(idx_hbm.at[pl.ds(base, _BPW)], idx_v)
    plsc.subcore_barrier()

    for j in range(_NCHUNK):
        for h in range(_CH // 16):
            vec = idx_v[pl.ds(j * _CH + h * 16, 16)]
            for i in range(16):
                pltpu.async_copy(
                    table_sh.at[vec[i]],
                    out_hbm.at[base + j * _CH + h * 16 + i],
                    sem)
    pltpu.make_async_copy(
        out_hbm.at[pl.ds(base, _BPW)], out_hbm.at[pl.ds(base, _BPW)],
        sem).wait()


def kernel(ids, indicator_embs):
    ids_flat = ids.reshape(_B).astype(jnp.int32)
    out = _sc_gather(ids_flat, indicator_embs)
    return out.reshape(ids.shape + (_HIDDEN,))


# speedup vs baseline: 1.1216x; 1.1216x over previous
import functools

import jax
import jax.numpy as jnp
from jax import lax
from jax.experimental import pallas as pl
from jax.experimental.pallas import tpu as pltpu
from jax.experimental.pallas import tpu_sc as plsc

_NROWS = 8
_HIDDEN = 1024
_B = 4 * 4096
_NC, _NS = 2, 16
_NW = _NC * _NS
_BPW = _B // _NW
_CH = 32
_NCHUNK = _BPW // _CH

_mesh = plsc.VectorSubcoreMesh(core_axis_name="c", subcore_axis_name="s")


@functools.partial(
    pl.kernel,
    mesh=_mesh,
    out_type=jax.ShapeDtypeStruct((_B, _HIDDEN), jnp.float32),
    scratch_types=[
        pltpu.VMEM((_BPW,), jnp.int32),
        pltpu.VMEM_SHARED((4 * _NROWS, _HIDDEN), jnp.float32),
        pltpu.SemaphoreType.DMA,
    ],
)
def _sc_gather(idx_hbm, table_hbm, out_hbm, idx_v, table_sh, sem):
    wid = lax.axis_index("s") * _NC + lax.axis_index("c")
    base = wid * _BPW

    @pl.when(lax.axis_index("s") == 0)
    def _():
        for k in range(4):
            pltpu.sync_copy(table_hbm, table_sh.at[pl.ds(k * _NROWS, _NROWS)])

    pltpu.sync_copy(idx_hbm.at[pl.ds(base, _BPW)], idx_v)
    plsc.subcore_barrier()
    bias = (lax.axis_index("s") % 4) * _NROWS

    def chunk(j, carry):
        for h in range(_CH // 16):
            vec = idx_v[pl.ds(j * _CH + h * 16, 16)] + bias
            for i in range(16):
                pltpu.async_copy(
                    table_sh.at[vec[i]],
                    out_hbm.at[base + j * _CH + h * 16 + i],
                    sem)
        return carry

    lax.fori_loop(0, _NCHUNK, chunk, 0)
    pltpu.make_async_copy(
        out_hbm.at[pl.ds(base, _BPW)], out_hbm.at[pl.ds(base, _BPW)],
        sem).wait()


def kernel(ids, indicator_embs):
    ids_flat = ids.reshape(_B).astype(jnp.int32)
    out = _sc_gather(ids_flat, indicator_embs)
    return out.reshape(ids.shape + (_HIDDEN,))


# speedup vs baseline: 1.1793x; 1.0515x over previous
import functools

import jax
import jax.numpy as jnp
from jax import lax
from jax.experimental import pallas as pl
from jax.experimental.pallas import tpu as pltpu
from jax.experimental.pallas import tpu_sc as plsc

_NROWS = 8
_HIDDEN = 1024
_B = 4 * 4096
_NC, _NS = 2, 16
_NW = _NC * _NS
_BPW = _B // _NW
_CH = 32
_NCHUNK = _BPW // _CH

_mesh = plsc.VectorSubcoreMesh(core_axis_name="c", subcore_axis_name="s")


@functools.partial(
    pl.kernel,
    mesh=_mesh,
    out_type=jax.ShapeDtypeStruct((_B, _HIDDEN), jnp.float32),
    scratch_types=[
        pltpu.VMEM((_BPW,), jnp.int32),
        pltpu.VMEM_SHARED((_NROWS, _HIDDEN), jnp.float32),
        pltpu.SemaphoreType.DMA,
    ],
)
def _sc_gather(idx_hbm, table_hbm, out_hbm, idx_v, table_sh, sem):
    wid = lax.axis_index("s") * _NC + lax.axis_index("c")
    base = wid * _BPW

    @pl.when(lax.axis_index("s") == 0)
    def _():
        pltpu.sync_copy(table_hbm, table_sh)

    pltpu.sync_copy(idx_hbm.at[pl.ds(base, _BPW)], idx_v)
    plsc.subcore_barrier()

    def chunk(j, carry):
        for h in range(_CH // 16):
            vec = idx_v[pl.ds(j * _CH + h * 16, 16)]
            for i in range(16):
                pltpu.async_copy(
                    table_sh.at[vec[i]],
                    out_hbm.at[base + j * _CH + h * 16 + i],
                    sem)
        return carry

    lax.fori_loop(0, _NCHUNK, chunk, 0)
    pltpu.make_async_copy(
        out_hbm.at[pl.ds(base, _BPW)], out_hbm.at[pl.ds(base, _BPW)],
        sem).wait()


def kernel(ids, indicator_embs):
    ids_flat = ids.reshape(_B).astype(jnp.int32)
    out = _sc_gather(ids_flat, indicator_embs)
    return out.reshape(ids.shape + (_HIDDEN,))
